# Initial kernel scaffold; baseline (speedup 1.0000x reference)
import functools

import jax
import jax.numpy as jnp
from jax import lax
from jax.experimental import pallas as pl
from jax.experimental.pallas import tpu as pltpu
from jax.experimental.pallas import tpu_sc as plsc

N_NODES = 10000
N_EDGES = 320000
D_FEAT = 128
O_OUT = 64

NC = 2
NS = 16
LANES = 128
IB = 2528
BLOCKS_PER_TILE = IB // (NC * NS)
N_ACC = 10016
ROWS_PER_TILE = N_ACC // NS
DEG_W = 16


def _agg_body(compute_deg, *refs):
    if compute_deg:
        (src_hbm, rc_hbm, p_hbm, degp_hbm,
         rc_v, rows_v, ones_v, acc, dacc, sem) = refs
    else:
        (src_hbm, rc_hbm, p_hbm,
         rc_v, rows_v, acc, sem) = refs

    core = lax.axis_index("core")
    sub = lax.axis_index("subcore")

    @pl.loop(0, LANES)
    def _(r):
        @pl.loop(0, D_FEAT // 16)
        def _(j):
            rows_v[r, pl.ds(pl.multiple_of(j * 16, 16), 16)] = jnp.zeros(
                (16,), jnp.float32)

    if compute_deg:
        @pl.loop(0, LANES)
        def _(r):
            ones_v[r, :] = jnp.zeros((16,), jnp.float32)

    r0 = sub * ROWS_PER_TILE
    for k in range(4):
        pltpu.sync_copy(rows_v,
                        acc.at[pl.ds(r0 + k * LANES, LANES)])
    pltpu.sync_copy(rows_v.at[pl.ds(0, ROWS_PER_TILE - 4 * LANES)],
                    acc.at[pl.ds(r0 + 4 * LANES, ROWS_PER_TILE - 4 * LANES)])
    if compute_deg:
        for k in range(4):
            pltpu.sync_copy(ones_v,
                            dacc.at[pl.ds(r0 + k * LANES, LANES)])
        pltpu.sync_copy(ones_v.at[pl.ds(0, ROWS_PER_TILE - 4 * LANES)],
                        dacc.at[pl.ds(r0 + 4 * LANES,
                                      ROWS_PER_TILE - 4 * LANES)])
        @pl.loop(0, LANES)
        def _(r):
            ones_v[r, :] = jnp.ones((16,), jnp.float32)

    plsc.subcore_barrier()

    base = (core * NS + sub) * BLOCKS_PER_TILE

    @pl.loop(0, BLOCKS_PER_TILE)
    def _(k):
        blk = base + k
        pltpu.sync_copy(rc_hbm.at[pl.ds(blk, 1)], rc_v)
        row_idx = rc_v.at[0, 0]
        col_idx = rc_v.at[0, 1]
        pltpu.async_copy(src_hbm.at[row_idx], rows_v, sem).wait()
        pltpu.sync_copy(rows_v, acc.at[col_idx], add=True)
        if compute_deg:
            pltpu.sync_copy(ones_v, dacc.at[col_idx], add=True)

    plsc.subcore_barrier()

    pltpu.sync_copy(acc.at[pl.ds(r0, ROWS_PER_TILE)],
                    p_hbm.at[core, pl.ds(r0, ROWS_PER_TILE)])
    if compute_deg:
        pltpu.sync_copy(dacc.at[pl.ds(r0, ROWS_PER_TILE)],
                        degp_hbm.at[core, pl.ds(r0, ROWS_PER_TILE)])


def _sc_aggregate(src, rc, compute_deg):
    mesh = plsc.VectorSubcoreMesh(core_axis_name="core",
                                  subcore_axis_name="subcore")
    outs = [jax.ShapeDtypeStruct((NC, N_ACC, D_FEAT), jnp.float32)]
    scratch = [pltpu.VMEM((1, 2, LANES), jnp.int32),
               pltpu.VMEM((LANES, D_FEAT), jnp.float32)]
    if compute_deg:
        outs.append(jax.ShapeDtypeStruct((NC, N_ACC, DEG_W), jnp.float32))
        scratch.append(pltpu.VMEM((LANES, DEG_W), jnp.float32))
    scratch.append(pltpu.VMEM_SHARED((N_ACC, D_FEAT), jnp.float32))
    if compute_deg:
        scratch.append(pltpu.VMEM_SHARED((N_ACC, DEG_W), jnp.float32))
    scratch.append(pltpu.SemaphoreType.DMA)
    body = functools.partial(_agg_body, compute_deg)
    out = pl.kernel(body, out_type=tuple(outs), mesh=mesh,
                    scratch_types=scratch)(src, rc)
    return out


ROW_BLK = 1000


def _combine_body(p_ref, degp_ref, n1_ref):
    deg = degp_ref[0] + degp_ref[1]
    dinv = 1.0 / jnp.maximum(deg[:, :1], 1.0)
    n1_ref[...] = (p_ref[0] + p_ref[1]) * dinv


def _combine(p, degp):
    grid = (N_NODES // ROW_BLK,)
    return pl.pallas_call(
        _combine_body,
        grid=grid,
        in_specs=[
            pl.BlockSpec((NC, ROW_BLK, D_FEAT), lambda i: (0, i, 0)),
            pl.BlockSpec((NC, ROW_BLK, DEG_W), lambda i: (0, i, 0)),
        ],
        out_specs=pl.BlockSpec((ROW_BLK, D_FEAT), lambda i: (i, 0)),
        out_shape=jax.ShapeDtypeStruct((N_NODES, D_FEAT), jnp.float32),
    )(p, degp)


def _final_body(x_ref, n1_ref, q_ref, degp_ref, m_ref, bc_ref, wo_ref,
                bo_ref, out_ref):
    deg = degp_ref[0] + degp_ref[1]
    dinv = 1.0 / jnp.maximum(deg[:, :1], 1.0)
    n2 = (q_ref[0] + q_ref[1]) * dinv
    h = jnp.dot(x_ref[...], m_ref[0], preferred_element_type=jnp.float32)
    h = h + jnp.dot(n1_ref[...], m_ref[1], preferred_element_type=jnp.float32)
    h = h + jnp.dot(n2, m_ref[2], preferred_element_type=jnp.float32)
    h = jnp.maximum(h + bc_ref[...], 0.0)
    out_ref[...] = (jnp.dot(h, wo_ref[...], preferred_element_type=jnp.float32)
                    + bo_ref[...])


def _final(x, n1, q, degp, m, bc, wo, bo):
    grid = (N_NODES // ROW_BLK,)
    return pl.pallas_call(
        _final_body,
        grid=grid,
        in_specs=[
            pl.BlockSpec((ROW_BLK, D_FEAT), lambda i: (i, 0)),
            pl.BlockSpec((ROW_BLK, D_FEAT), lambda i: (i, 0)),
            pl.BlockSpec((NC, ROW_BLK, D_FEAT), lambda i: (0, i, 0)),
            pl.BlockSpec((NC, ROW_BLK, DEG_W), lambda i: (0, i, 0)),
            pl.BlockSpec((3, D_FEAT, D_FEAT), lambda i: (0, 0, 0)),
            pl.BlockSpec((1, D_FEAT), lambda i: (0, 0)),
            pl.BlockSpec((D_FEAT, O_OUT), lambda i: (0, 0)),
            pl.BlockSpec((1, O_OUT), lambda i: (0, 0)),
        ],
        out_specs=pl.BlockSpec((ROW_BLK, O_OUT), lambda i: (i, 0)),
        out_shape=jax.ShapeDtypeStruct((N_NODES, O_OUT), jnp.float32),
    )(x, n1, q, degp, m, bc, wo, bo)


def kernel(x, edge_index, W_ego, b_ego, W_n1, b_n1, W_n2, b_n2,
           W_comb, b_comb, W_out, b_out):
    row = edge_index[0]
    col = edge_index[1]

    pad = IB * LANES - N_EDGES
    ar = jnp.arange(pad, dtype=jnp.int32)
    row_p = jnp.concatenate([row, ar % N_NODES]).reshape(IB, LANES)
    col_p = jnp.concatenate([col, N_NODES + (ar % (N_ACC - N_NODES))]
                            ).reshape(IB, LANES)
    rc = jnp.stack([row_p, col_p], axis=1)

    m = jnp.stack([W_ego @ W_comb[:D_FEAT],
                   W_n1 @ W_comb[D_FEAT:2 * D_FEAT],
                   W_n2 @ W_comb[2 * D_FEAT:]], axis=0)
    bc = (b_ego @ W_comb[:D_FEAT] + b_n1 @ W_comb[D_FEAT:2 * D_FEAT]
          + b_n2 @ W_comb[2 * D_FEAT:] + b_comb)[None, :]

    p, degp = _sc_aggregate(x, rc, compute_deg=True)
    n1 = _combine(p, degp)
    q, = _sc_aggregate(n1, rc, compute_deg=False)
    return _final(x, n1, q, degp, m, bc, W_out, b_out[None, :])


# speedup vs baseline: 6.5067x; 6.5067x over previous
import functools

import jax
import jax.numpy as jnp
from jax import lax
from jax.experimental import pallas as pl
from jax.experimental.pallas import tpu as pltpu
from jax.experimental.pallas import tpu_sc as plsc

N_NODES = 10000
N_EDGES = 320000
D_FEAT = 128
O_OUT = 64

NC = 2
NS = 16
LANES = 128
GRP = 8
IB = 2560
BLOCKS_PER_TILE = IB // (NC * NS)
GROUPS_PER_TILE = BLOCKS_PER_TILE // GRP
N_ACC = 10112
ROWS_PER_TILE = N_ACC // NS
DEG_W = 16
AUG_W = D_FEAT + DEG_W


def _agg_body(width, *refs):
    (src_hbm, row_hbm, col_hbm, p_hbm,
     row_v, col_v, rows_v, acc, sem) = refs

    core = lax.axis_index("core")
    sub = lax.axis_index("subcore")

    @pl.loop(0, LANES)
    def _(r):
        @pl.loop(0, width // 16)
        def _(j):
            rows_v[r, pl.ds(pl.multiple_of(j * 16, 16), 16)] = jnp.zeros(
                (16,), jnp.float32)

    r0 = sub * ROWS_PER_TILE
    tail = ROWS_PER_TILE - 4 * LANES
    for k in range(4):
        pltpu.sync_copy(rows_v, acc.at[pl.ds(r0 + k * LANES, LANES)])
    pltpu.sync_copy(rows_v.at[pl.ds(0, tail)],
                    acc.at[pl.ds(r0 + 4 * LANES, tail)])

    plsc.subcore_barrier()

    base = (core * NS + sub) * BLOCKS_PER_TILE

    @pl.loop(0, GROUPS_PER_TILE)
    def _(g):
        blk = base + g * GRP
        pltpu.sync_copy(row_hbm.at[pl.ds(blk, GRP)], row_v)
        pltpu.sync_copy(col_hbm.at[pl.ds(blk, GRP)], col_v)
        for j in range(GRP):
            pltpu.async_copy(src_hbm.at[row_v.at[j]], rows_v, sem).wait()
            pltpu.sync_copy(rows_v, acc.at[col_v.at[j]], add=True)

    plsc.subcore_barrier()

    for k in range(4):
        pltpu.sync_copy(acc.at[pl.ds(r0 + k * LANES, LANES)], rows_v)
        pltpu.sync_copy(rows_v, p_hbm.at[core, pl.ds(r0 + k * LANES, LANES)])
    pltpu.sync_copy(acc.at[pl.ds(r0 + 4 * LANES, tail)],
                    rows_v.at[pl.ds(0, tail)])
    pltpu.sync_copy(rows_v.at[pl.ds(0, tail)],
                    p_hbm.at[core, pl.ds(r0 + 4 * LANES, tail)])


def _sc_aggregate(src, row2, col2, width):
    mesh = plsc.VectorSubcoreMesh(core_axis_name="core",
                                  subcore_axis_name="subcore")
    body = functools.partial(_agg_body, width)
    cp = None
    if width % 128 != 0:
        cp = pltpu.CompilerParams(use_tc_tiling_on_sc=False)
    return pl.kernel(
        body, mesh=mesh, compiler_params=cp,
        out_type=jax.ShapeDtypeStruct((NC, N_ACC, width), jnp.float32),
        scratch_types=[pltpu.VMEM((GRP, LANES), jnp.int32),
                       pltpu.VMEM((GRP, LANES), jnp.int32),
                       pltpu.VMEM((LANES, width), jnp.float32),
                       pltpu.VMEM_SHARED((N_ACC, width), jnp.float32),
                       pltpu.SemaphoreType.DMA])(src, row2, col2)


ROW_BLK = 1000


def _combine_body(p_ref, n1_ref):
    deg = p_ref[0, :, D_FEAT:D_FEAT + 1] + p_ref[1, :, D_FEAT:D_FEAT + 1]
    dinv = 1.0 / jnp.maximum(deg, 1.0)
    n1_ref[...] = (p_ref[0, :, :D_FEAT] + p_ref[1, :, :D_FEAT]) * dinv


def _combine(p):
    grid = (N_NODES // ROW_BLK,)
    return pl.pallas_call(
        _combine_body,
        grid=grid,
        in_specs=[
            pl.BlockSpec((NC, ROW_BLK, AUG_W), lambda i: (0, i, 0)),
        ],
        out_specs=pl.BlockSpec((ROW_BLK, D_FEAT), lambda i: (i, 0)),
        out_shape=jax.ShapeDtypeStruct((N_NODES, D_FEAT), jnp.float32),
    )(p)


def _final_body(x_ref, n1_ref, q_ref, degp_ref, m_ref, bc_ref, wo_ref,
                bo_ref, out_ref):
    deg = (degp_ref[0, :, D_FEAT:D_FEAT + 1]
           + degp_ref[1, :, D_FEAT:D_FEAT + 1])
    dinv = 1.0 / jnp.maximum(deg, 1.0)
    n2 = (q_ref[0] + q_ref[1]) * dinv
    h = jnp.dot(x_ref[...], m_ref[0], preferred_element_type=jnp.float32)
    h = h + jnp.dot(n1_ref[...], m_ref[1], preferred_element_type=jnp.float32)
    h = h + jnp.dot(n2, m_ref[2], preferred_element_type=jnp.float32)
    h = jnp.maximum(h + bc_ref[...], 0.0)
    out_ref[...] = (jnp.dot(h, wo_ref[...], preferred_element_type=jnp.float32)
                    + bo_ref[...])


def _final(x, n1, q, p, m, bc, wo, bo):
    grid = (N_NODES // ROW_BLK,)
    return pl.pallas_call(
        _final_body,
        grid=grid,
        in_specs=[
            pl.BlockSpec((ROW_BLK, D_FEAT), lambda i: (i, 0)),
            pl.BlockSpec((ROW_BLK, D_FEAT), lambda i: (i, 0)),
            pl.BlockSpec((NC, ROW_BLK, D_FEAT), lambda i: (0, i, 0)),
            pl.BlockSpec((NC, ROW_BLK, AUG_W), lambda i: (0, i, 0)),
            pl.BlockSpec((3, D_FEAT, D_FEAT), lambda i: (0, 0, 0)),
            pl.BlockSpec((1, D_FEAT), lambda i: (0, 0)),
            pl.BlockSpec((D_FEAT, O_OUT), lambda i: (0, 0)),
            pl.BlockSpec((1, O_OUT), lambda i: (0, 0)),
        ],
        out_specs=pl.BlockSpec((ROW_BLK, O_OUT), lambda i: (i, 0)),
        out_shape=jax.ShapeDtypeStruct((N_NODES, O_OUT), jnp.float32),
    )(x, n1, q, p, m, bc, wo, bo)


def kernel(x, edge_index, W_ego, b_ego, W_n1, b_n1, W_n2, b_n2,
           W_comb, b_comb, W_out, b_out):
    row = edge_index[0]
    col = edge_index[1]

    pad = IB * LANES - N_EDGES
    ar = jnp.arange(pad, dtype=jnp.int32)
    row2 = jnp.concatenate([row, ar % N_NODES]).reshape(IB, LANES)
    col2 = jnp.concatenate([col, N_NODES + (ar % (N_ACC - N_NODES))]
                           ).reshape(IB, LANES)

    xa = jnp.concatenate(
        [x, jnp.ones((N_NODES, DEG_W), dtype=jnp.float32)], axis=1)

    m = jnp.stack([W_ego @ W_comb[:D_FEAT],
                   W_n1 @ W_comb[D_FEAT:2 * D_FEAT],
                   W_n2 @ W_comb[2 * D_FEAT:]], axis=0)
    bc = (b_ego @ W_comb[:D_FEAT] + b_n1 @ W_comb[D_FEAT:2 * D_FEAT]
          + b_n2 @ W_comb[2 * D_FEAT:] + b_comb)[None, :]

    p = _sc_aggregate(xa, row2, col2, width=AUG_W)
    n1 = _combine(p)
    q = _sc_aggregate(n1, row2, col2, width=D_FEAT)
    return _final(x, n1, q, p, m, bc, W_out, b_out[None, :])


# speedup vs baseline: 8.4604x; 1.3003x over previous
import functools

import jax
import jax.numpy as jnp
from jax import lax
from jax.experimental import pallas as pl
from jax.experimental.pallas import tpu as pltpu
from jax.experimental.pallas import tpu_sc as plsc

N_NODES = 10000
N_EDGES = 320000
D_FEAT = 128
O_OUT = 64

NC = 2
NS = 16
NW = NC * NS
GRP = 16
E_PAD = 327680
N_ACC = 10112
ROWS_PER_TILE = N_ACC // NS
DEG_W = 16
AUG_W = D_FEAT + DEG_W


def _agg_body(width, wlen, *refs):
    (src_hbm, row_hbm, col_hbm, p_hbm,
     row_v, col_v, rows_a, rows_b, acc, sem_a, sem_b) = refs

    bufs = (rows_a, rows_b)
    sems = (sem_a, sem_b)
    core = lax.axis_index("core")
    sub = lax.axis_index("subcore")
    wpt = E_PAD // wlen // NW
    grps = wpt // GRP

    @pl.loop(0, wlen)
    def _(r):
        @pl.loop(0, width // 16)
        def _(j):
            rows_a[r, pl.ds(pl.multiple_of(j * 16, 16), 16)] = jnp.zeros(
                (16,), jnp.float32)

    r0 = sub * ROWS_PER_TILE
    chunks = ROWS_PER_TILE // wlen
    tail = ROWS_PER_TILE - chunks * wlen
    for k in range(chunks):
        pltpu.sync_copy(rows_a, acc.at[pl.ds(r0 + k * wlen, wlen)])
    pltpu.sync_copy(rows_a.at[pl.ds(0, tail)],
                    acc.at[pl.ds(r0 + chunks * wlen, tail)])

    plsc.subcore_barrier()

    base = (core * NS + sub) * wpt

    @pl.loop(0, grps)
    def _(g):
        blk = base + g * GRP
        pltpu.sync_copy(row_hbm.at[pl.ds(blk, GRP)], row_v)
        pltpu.sync_copy(col_hbm.at[pl.ds(blk, GRP)], col_v)
        copies = [pltpu.async_copy(src_hbm.at[row_v.at[0]], bufs[0], sems[0]),
                  None]
        for j in range(1, GRP):
            b = j % 2
            copies[b] = pltpu.async_copy(src_hbm.at[row_v.at[j]],
                                         bufs[b], sems[b])
            copies[1 - b].wait()
            pltpu.sync_copy(bufs[1 - b], acc.at[col_v.at[j - 1]], add=True)
        last = (GRP - 1) % 2
        copies[last].wait()
        pltpu.sync_copy(bufs[last], acc.at[col_v.at[GRP - 1]], add=True)

    plsc.subcore_barrier()

    for k in range(chunks):
        pltpu.sync_copy(acc.at[pl.ds(r0 + k * wlen, wlen)], rows_a)
        pltpu.sync_copy(rows_a, p_hbm.at[core, pl.ds(r0 + k * wlen, wlen)])
    pltpu.sync_copy(acc.at[pl.ds(r0 + chunks * wlen, tail)],
                    rows_a.at[pl.ds(0, tail)])
    pltpu.sync_copy(rows_a.at[pl.ds(0, tail)],
                    p_hbm.at[core, pl.ds(r0 + chunks * wlen, tail)])


def _sc_aggregate(src, row2, col2, width, wlen):
    mesh = plsc.VectorSubcoreMesh(core_axis_name="core",
                                  subcore_axis_name="subcore")
    body = functools.partial(_agg_body, width, wlen)
    cp = None
    if width % 128 != 0:
        cp = pltpu.CompilerParams(use_tc_tiling_on_sc=False)
    return pl.kernel(
        body, mesh=mesh, compiler_params=cp,
        out_type=jax.ShapeDtypeStruct((NC, N_ACC, width), jnp.float32),
        scratch_types=[pltpu.VMEM((GRP, wlen), jnp.int32),
                       pltpu.VMEM((GRP, wlen), jnp.int32),
                       pltpu.VMEM((wlen, width), jnp.float32),
                       pltpu.VMEM((wlen, width), jnp.float32),
                       pltpu.VMEM_SHARED((N_ACC, width), jnp.float32),
                       pltpu.SemaphoreType.DMA,
                       pltpu.SemaphoreType.DMA])(src, row2, col2)


ROW_BLK = 1000


def _combine_body(p_ref, n1_ref):
    deg = p_ref[0, :, D_FEAT:D_FEAT + 1] + p_ref[1, :, D_FEAT:D_FEAT + 1]
    dinv = 1.0 / jnp.maximum(deg, 1.0)
    n1_ref[...] = (p_ref[0, :, :D_FEAT] + p_ref[1, :, :D_FEAT]) * dinv


def _combine(p):
    grid = (N_NODES // ROW_BLK,)
    return pl.pallas_call(
        _combine_body,
        grid=grid,
        in_specs=[
            pl.BlockSpec((NC, ROW_BLK, AUG_W), lambda i: (0, i, 0)),
        ],
        out_specs=pl.BlockSpec((ROW_BLK, D_FEAT), lambda i: (i, 0)),
        out_shape=jax.ShapeDtypeStruct((N_NODES, D_FEAT), jnp.float32),
    )(p)


def _final_body(x_ref, n1_ref, q_ref, degp_ref, m_ref, bc_ref, wo_ref,
                bo_ref, out_ref):
    deg = (degp_ref[0, :, D_FEAT:D_FEAT + 1]
           + degp_ref[1, :, D_FEAT:D_FEAT + 1])
    dinv = 1.0 / jnp.maximum(deg, 1.0)
    n2 = (q_ref[0] + q_ref[1]) * dinv
    h = jnp.dot(x_ref[...], m_ref[0], preferred_element_type=jnp.float32)
    h = h + jnp.dot(n1_ref[...], m_ref[1], preferred_element_type=jnp.float32)
    h = h + jnp.dot(n2, m_ref[2], preferred_element_type=jnp.float32)
    h = jnp.maximum(h + bc_ref[...], 0.0)
    out_ref[...] = (jnp.dot(h, wo_ref[...], preferred_element_type=jnp.float32)
                    + bo_ref[...])


def _final(x, n1, q, p, m, bc, wo, bo):
    grid = (N_NODES // ROW_BLK,)
    return pl.pallas_call(
        _final_body,
        grid=grid,
        in_specs=[
            pl.BlockSpec((ROW_BLK, D_FEAT), lambda i: (i, 0)),
            pl.BlockSpec((ROW_BLK, D_FEAT), lambda i: (i, 0)),
            pl.BlockSpec((NC, ROW_BLK, D_FEAT), lambda i: (0, i, 0)),
            pl.BlockSpec((NC, ROW_BLK, AUG_W), lambda i: (0, i, 0)),
            pl.BlockSpec((3, D_FEAT, D_FEAT), lambda i: (0, 0, 0)),
            pl.BlockSpec((1, D_FEAT), lambda i: (0, 0)),
            pl.BlockSpec((D_FEAT, O_OUT), lambda i: (0, 0)),
            pl.BlockSpec((1, O_OUT), lambda i: (0, 0)),
        ],
        out_specs=pl.BlockSpec((ROW_BLK, O_OUT), lambda i: (i, 0)),
        out_shape=jax.ShapeDtypeStruct((N_NODES, O_OUT), jnp.float32),
    )(x, n1, q, p, m, bc, wo, bo)


def kernel(x, edge_index, W_ego, b_ego, W_n1, b_n1, W_n2, b_n2,
           W_comb, b_comb, W_out, b_out):
    row = edge_index[0]
    col = edge_index[1]

    pad = E_PAD - N_EDGES
    ar = jnp.arange(pad, dtype=jnp.int32)
    rowp = jnp.concatenate([row, ar % N_NODES])
    colp = jnp.concatenate([col, N_NODES + (ar % (N_ACC - N_NODES))])

    xa = jnp.concatenate(
        [x, jnp.ones((N_NODES, DEG_W), dtype=jnp.float32)], axis=1)

    m = jnp.stack([W_ego @ W_comb[:D_FEAT],
                   W_n1 @ W_comb[D_FEAT:2 * D_FEAT],
                   W_n2 @ W_comb[2 * D_FEAT:]], axis=0)
    bc = (b_ego @ W_comb[:D_FEAT] + b_n1 @ W_comb[D_FEAT:2 * D_FEAT]
          + b_n2 @ W_comb[2 * D_FEAT:] + b_comb)[None, :]

    p = _sc_aggregate(xa, rowp.reshape(-1, 64), colp.reshape(-1, 64),
                      width=AUG_W, wlen=64)
    n1 = _combine(p)
    q = _sc_aggregate(n1, rowp.reshape(-1, 128), colp.reshape(-1, 128),
                      width=D_FEAT, wlen=128)
    return _final(x, n1, q, p, m, bc, W_out, b_out[None, :])


# speedup vs baseline: 8.7036x; 1.0287x over previous
import functools

import jax
import jax.numpy as jnp
from jax import lax
from jax.experimental import pallas as pl
from jax.experimental.pallas import tpu as pltpu
from jax.experimental.pallas import tpu_sc as plsc

N_NODES = 10000
N_EDGES = 320000
D_FEAT = 128
O_OUT = 64

NC = 2
NS = 16
NW = NC * NS
GRP = 16
E_PAD = 327680
N_ACC = 10112
ROWS_PER_TILE = N_ACC // NS
DEG_W = 16
AUG_W = D_FEAT + DEG_W


def _agg_body(width, wlen, *refs):
    (src_hbm, row_hbm, col_hbm, p_hbm,
     row_v, col_v, rows_a, rows_b, acc, sem_a, sem_b) = refs

    bufs = (rows_a, rows_b)
    sems = (sem_a, sem_b)
    core = lax.axis_index("core")
    sub = lax.axis_index("subcore")
    wpt = E_PAD // wlen // NW
    grps = wpt // GRP

    @pl.loop(0, wlen)
    def _(r):
        @pl.loop(0, width // 16)
        def _(j):
            rows_a[r, pl.ds(pl.multiple_of(j * 16, 16), 16)] = jnp.zeros(
                (16,), jnp.float32)

    r0 = sub * ROWS_PER_TILE
    chunks = ROWS_PER_TILE // wlen
    tail = ROWS_PER_TILE - chunks * wlen
    for k in range(chunks):
        pltpu.sync_copy(rows_a, acc.at[pl.ds(r0 + k * wlen, wlen)])
    pltpu.sync_copy(rows_a.at[pl.ds(0, tail)],
                    acc.at[pl.ds(r0 + chunks * wlen, tail)])

    plsc.subcore_barrier()

    base = (core * NS + sub) * wpt

    @pl.loop(0, grps)
    def _(g):
        blk = base + g * GRP
        pltpu.sync_copy(row_hbm.at[pl.ds(blk, GRP)], row_v)
        pltpu.sync_copy(col_hbm.at[pl.ds(blk, GRP)], col_v)
        copies = [pltpu.async_copy(src_hbm.at[row_v.at[0]], bufs[0], sems[0]),
                  None]
        for j in range(1, GRP):
            b = j % 2
            copies[b] = pltpu.async_copy(src_hbm.at[row_v.at[j]],
                                         bufs[b], sems[b])
            copies[1 - b].wait()
            pltpu.sync_copy(bufs[1 - b], acc.at[col_v.at[j - 1]], add=True)
        last = (GRP - 1) % 2
        copies[last].wait()
        pltpu.sync_copy(bufs[last], acc.at[col_v.at[GRP - 1]], add=True)

    plsc.subcore_barrier()

    for k in range(chunks):
        pltpu.sync_copy(acc.at[pl.ds(r0 + k * wlen, wlen)], rows_a)
        pltpu.sync_copy(rows_a, p_hbm.at[core, pl.ds(r0 + k * wlen, wlen)])
    pltpu.sync_copy(acc.at[pl.ds(r0 + chunks * wlen, tail)],
                    rows_a.at[pl.ds(0, tail)])
    pltpu.sync_copy(rows_a.at[pl.ds(0, tail)],
                    p_hbm.at[core, pl.ds(r0 + chunks * wlen, tail)])


def _sc_aggregate(src, row2, col2, width, wlen):
    mesh = plsc.VectorSubcoreMesh(core_axis_name="core",
                                  subcore_axis_name="subcore")
    body = functools.partial(_agg_body, width, wlen)
    cp = None
    if width % 128 != 0:
        cp = pltpu.CompilerParams(use_tc_tiling_on_sc=False)
    return pl.kernel(
        body, mesh=mesh, compiler_params=cp,
        out_type=jax.ShapeDtypeStruct((NC, N_ACC, width), jnp.float32),
        scratch_types=[pltpu.VMEM((GRP, wlen), jnp.int32),
                       pltpu.VMEM((GRP, wlen), jnp.int32),
                       pltpu.VMEM((wlen, width), jnp.float32),
                       pltpu.VMEM((wlen, width), jnp.float32),
                       pltpu.VMEM_SHARED((N_ACC, width), jnp.float32),
                       pltpu.SemaphoreType.DMA,
                       pltpu.SemaphoreType.DMA])(src, row2, col2)


ROW_BLK = 1000


def _combine_body(p_ref, n1_ref):
    deg = p_ref[0, :, D_FEAT:D_FEAT + 1] + p_ref[1, :, D_FEAT:D_FEAT + 1]
    dinv = 1.0 / jnp.maximum(deg, 1.0)
    n1_ref[...] = (p_ref[0, :, :D_FEAT] + p_ref[1, :, :D_FEAT]) * dinv


def _combine(p):
    grid = (N_NODES // ROW_BLK,)
    return pl.pallas_call(
        _combine_body,
        grid=grid,
        in_specs=[
            pl.BlockSpec((NC, ROW_BLK, AUG_W), lambda i: (0, i, 0)),
        ],
        out_specs=pl.BlockSpec((ROW_BLK, D_FEAT), lambda i: (i, 0)),
        out_shape=jax.ShapeDtypeStruct((N_NODES, D_FEAT), jnp.float32),
    )(p)


def _ego_body(x_ref, n1_ref, m_ref, bc_ref, y_ref):
    h = jnp.dot(x_ref[...], m_ref[0], preferred_element_type=jnp.float32)
    h = h + jnp.dot(n1_ref[...], m_ref[1], preferred_element_type=jnp.float32)
    y_ref[...] = h + bc_ref[...]


def _ego(x, n1, m, bc):
    grid = (N_NODES // ROW_BLK,)
    return pl.pallas_call(
        _ego_body,
        grid=grid,
        in_specs=[
            pl.BlockSpec((ROW_BLK, D_FEAT), lambda i: (i, 0)),
            pl.BlockSpec((ROW_BLK, D_FEAT), lambda i: (i, 0)),
            pl.BlockSpec((3, D_FEAT, D_FEAT), lambda i: (0, 0, 0)),
            pl.BlockSpec((1, D_FEAT), lambda i: (0, 0)),
        ],
        out_specs=pl.BlockSpec((ROW_BLK, D_FEAT), lambda i: (i, 0)),
        out_shape=jax.ShapeDtypeStruct((N_NODES, D_FEAT), jnp.float32),
    )(x, n1, m, bc)


def _final_body(y_ref, q_ref, degp_ref, m_ref, wo_ref, bo_ref, out_ref):
    deg = (degp_ref[0, :, D_FEAT:D_FEAT + 1]
           + degp_ref[1, :, D_FEAT:D_FEAT + 1])
    dinv = 1.0 / jnp.maximum(deg, 1.0)
    n2 = (q_ref[0] + q_ref[1]) * dinv
    h = y_ref[...] + jnp.dot(n2, m_ref[2], preferred_element_type=jnp.float32)
    h = jnp.maximum(h, 0.0)
    out_ref[...] = (jnp.dot(h, wo_ref[...], preferred_element_type=jnp.float32)
                    + bo_ref[...])


def _final(y, q, p, m, wo, bo):
    grid = (N_NODES // ROW_BLK,)
    return pl.pallas_call(
        _final_body,
        grid=grid,
        in_specs=[
            pl.BlockSpec((ROW_BLK, D_FEAT), lambda i: (i, 0)),
            pl.BlockSpec((NC, ROW_BLK, D_FEAT), lambda i: (0, i, 0)),
            pl.BlockSpec((NC, ROW_BLK, AUG_W), lambda i: (0, i, 0)),
            pl.BlockSpec((3, D_FEAT, D_FEAT), lambda i: (0, 0, 0)),
            pl.BlockSpec((D_FEAT, O_OUT), lambda i: (0, 0)),
            pl.BlockSpec((1, O_OUT), lambda i: (0, 0)),
        ],
        out_specs=pl.BlockSpec((ROW_BLK, O_OUT), lambda i: (i, 0)),
        out_shape=jax.ShapeDtypeStruct((N_NODES, O_OUT), jnp.float32),
    )(y, q, p, m, wo, bo)


def kernel(x, edge_index, W_ego, b_ego, W_n1, b_n1, W_n2, b_n2,
           W_comb, b_comb, W_out, b_out):
    row = edge_index[0]
    col = edge_index[1]

    pad = E_PAD - N_EDGES
    ar = jnp.arange(pad, dtype=jnp.int32)
    rowp = jnp.concatenate([row, ar % N_NODES])
    colp = jnp.concatenate([col, N_NODES + (ar % (N_ACC - N_NODES))])

    xa = jnp.concatenate(
        [x, jnp.ones((N_NODES, DEG_W), dtype=jnp.float32)], axis=1)

    m = jnp.stack([W_ego @ W_comb[:D_FEAT],
                   W_n1 @ W_comb[D_FEAT:2 * D_FEAT],
                   W_n2 @ W_comb[2 * D_FEAT:]], axis=0)
    bc = (b_ego @ W_comb[:D_FEAT] + b_n1 @ W_comb[D_FEAT:2 * D_FEAT]
          + b_n2 @ W_comb[2 * D_FEAT:] + b_comb)[None, :]

    p = _sc_aggregate(xa, rowp.reshape(-1, 80), colp.reshape(-1, 80),
                      width=AUG_W, wlen=80)
    n1 = _combine(p)
    q = _sc_aggregate(n1, rowp.reshape(-1, 128), colp.reshape(-1, 128),
                      width=D_FEAT, wlen=128)
    y = _ego(x, n1, m, bc)
    return _final(y, q, p, m, W_out, b_out[None, :])
